# baseline (speedup 1.0000x reference)
import functools

import jax
import jax.numpy as jnp
import numpy as np
from jax import lax
from jax.experimental import pallas as pl
from jax.experimental.pallas import tpu as pltpu
from jax.experimental.pallas import tpu_sc as plsc

N = 500000
NUM_SEMANTIC = 100000
DIM_SEMANTIC = 64
C_DIM = 128
NUM_FREQS = 5
MAX_FREQ_LOG2 = 4.0

NW = 32
CHUNK = 128
NCHUNK = 128
B_PER_W = CHUNK * NCHUNK
NPAD = NW * B_PER_W
NBUF = 4

BLK = 4096


def _sc_gather_fn():
    info = plsc.get_sparse_core_info()
    nc = info.num_cores

    mesh = plsc.VectorSubcoreMesh(core_axis_name="c", subcore_axis_name="s")

    @functools.partial(
        pl.kernel,
        mesh=mesh,
        compiler_params=pltpu.CompilerParams(use_tc_tiling_on_sc=True),
        out_type=jax.ShapeDtypeStruct((NPAD, C_DIM), jnp.float32),
        scratch_types=[
            pltpu.VMEM((NCHUNK, CHUNK), jnp.int32),
            pltpu.VMEM((NBUF, CHUNK, C_DIM), jnp.float32),
            pltpu.SemaphoreType.DMA((NBUF,)),
            pltpu.SemaphoreType.DMA((NBUF,)),
        ],
    )
    def sc_gather(ids_hbm, table_hbm, out_hbm, idx_v, rows_v, gsem, wsem):
        wid = lax.axis_index("s") * nc + lax.axis_index("c")
        base = wid * B_PER_W
        pltpu.sync_copy(ids_hbm.at[wid], idx_v)

        def gather(i, r):
            pltpu.async_copy(table_hbm.at[idx_v.at[i]], rows_v.at[r],
                             gsem.at[r])

        def wb_copy(i, r):
            return pltpu.make_async_copy(
                rows_v.at[r], out_hbm.at[pl.ds(base + i * CHUNK, CHUNK)],
                wsem.at[r])

        for i in range(NBUF - 1):
            gather(i, i)

        def body(i, carry):
            r = lax.rem(i, NBUF)
            r2 = lax.rem(i + NBUF - 1, NBUF)
            pltpu.make_async_copy(rows_v.at[r],
                                  out_hbm.at[pl.ds(base, CHUNK)],
                                  gsem.at[r]).wait()
            wb_copy(i, r).start()

            @pl.when(jnp.logical_and(i >= 1, i + NBUF - 1 < NCHUNK))
            def _():
                wb_copy(i, r2).wait()

            @pl.when(i + NBUF - 1 < NCHUNK)
            def _():
                gather(i + NBUF - 1, r2)

            return carry

        lax.fori_loop(0, NCHUNK, body, 0)
        for r in range(NBUF):
            pltpu.make_async_copy(rows_v.at[r],
                                  out_hbm.at[pl.ds(base, CHUNK)],
                                  wsem.at[r]).wait()

    return sc_gather


def _pos_weights():
    m = np.zeros((32, 3), np.float32)
    ph = np.zeros((32, 1), np.float32)
    freqs = 2.0 ** np.linspace(0.0, MAX_FREQ_LOG2, NUM_FREQS)
    for i in range(NUM_FREQS):
        for s in range(2):
            for c in range(3):
                k = 6 * i + 3 * s + c
                m[k, c] = freqs[i]
                ph[k, 0] = 0.0 if s == 0 else np.pi / 2.0
    return jnp.asarray(m), jnp.asarray(ph)


_SIN_C = (6.2831852819, -41.341698212, 81.605064899, -76.702152496,
          42.020491157, -14.883436519, 3.2191201543)
_INV_2PI = 0.15915493667125702


def _fast_sin(t):
    u = t * jnp.float32(_INV_2PI)
    r = u - jnp.floor(u + jnp.float32(0.5))
    r2 = r * r
    acc = jnp.float32(_SIN_C[6])
    for k in range(5, -1, -1):
        acc = acc * r2 + jnp.float32(_SIN_C[k])
    return acc * r


def _dot_t(a, b):
    return lax.dot_general(a, b, (((0,), (0,)), ((), ())),
                           preferred_element_type=jnp.float32)


def _tc_body(sem_ref, coords_ref, inten_ref, m2_ref, ph_ref, wsin_ref,
             wsem_ref, wraw_ref, wi_ref, b_ref, out_ref):
    xt = coords_ref[...]
    t = lax.dot_general(m2_ref[...], xt, (((1,), (0,)), ((), ())),
                        preferred_element_type=jnp.float32,
                        precision=lax.Precision.HIGHEST)
    t = t + ph_ref[...]
    s = _fast_sin(t)
    acc = _dot_t(s, wsin_ref[...])
    acc = acc + jnp.dot(sem_ref[...], wsem_ref[...],
                        preferred_element_type=jnp.float32)
    acc = acc + _dot_t(xt, wraw_ref[...])
    acc = acc + _dot_t(inten_ref[...], wi_ref[...])
    acc = acc + b_ref[...]
    out_ref[...] = acc


def kernel(coords, semantic_ids, intensity, embed_table, W, b):
    ids_pad = jnp.pad(semantic_ids.astype(jnp.int32), (0, NPAD - N))
    ids3 = ids_pad.reshape(NW, NCHUNK, CHUNK)
    table128 = jnp.pad(embed_table, ((0, 0), (0, C_DIM - DIM_SEMANTIC)))
    sem_g = _sc_gather_fn()(ids3, table128)

    coords_t = coords.T
    inten_t = intensity.T

    cols = W.T
    m2, ph = _pos_weights()
    wsin = jnp.zeros((32, C_DIM), jnp.float32).at[:30].set(cols[3:33])
    wsem = jnp.zeros((C_DIM, C_DIM), jnp.float32).at[:64].set(cols[33:97])
    wraw = cols[0:3]
    wi = cols[97:98]
    b2 = b.reshape(1, C_DIM)

    grid = (N + BLK - 1) // BLK
    out = pl.pallas_call(
        _tc_body,
        grid=(grid,),
        in_specs=[
            pl.BlockSpec((BLK, C_DIM), lambda i: (i, 0)),
            pl.BlockSpec((3, BLK), lambda i: (0, i)),
            pl.BlockSpec((1, BLK), lambda i: (0, i)),
            pl.BlockSpec((32, 3), lambda i: (0, 0)),
            pl.BlockSpec((32, 1), lambda i: (0, 0)),
            pl.BlockSpec((32, C_DIM), lambda i: (0, 0)),
            pl.BlockSpec((C_DIM, C_DIM), lambda i: (0, 0)),
            pl.BlockSpec((3, C_DIM), lambda i: (0, 0)),
            pl.BlockSpec((1, C_DIM), lambda i: (0, 0)),
            pl.BlockSpec((1, C_DIM), lambda i: (0, 0)),
        ],
        out_specs=pl.BlockSpec((BLK, C_DIM), lambda i: (i, 0)),
        out_shape=jax.ShapeDtypeStruct((N, C_DIM), jnp.float32),
    )(sem_g, coords_t, inten_t, m2, ph, wsin, wsem, wraw, wi, b2)
    return out


# speedup vs baseline: 1.6484x; 1.6484x over previous
import functools

import jax
import jax.numpy as jnp
import numpy as np
from jax import lax
from jax.experimental import pallas as pl
from jax.experimental.pallas import tpu as pltpu
from jax.experimental.pallas import tpu_sc as plsc

N = 500000
NUM_SEMANTIC = 100000
DIM_SEMANTIC = 64
C_DIM = 128
NUM_FREQS = 5
MAX_FREQ_LOG2 = 4.0

NW = 32
CHUNK = 128
NCHUNK = 128
B_PER_W = CHUNK * NCHUNK
NPAD = NW * B_PER_W
NBUF = 6

BLK = 4096


def _sc_gather_fn():
    info = plsc.get_sparse_core_info()
    nc = info.num_cores

    mesh = plsc.VectorSubcoreMesh(core_axis_name="c", subcore_axis_name="s")

    @functools.partial(
        pl.kernel,
        mesh=mesh,
        compiler_params=pltpu.CompilerParams(use_tc_tiling_on_sc=False),
        out_type=jax.ShapeDtypeStruct((NPAD, C_DIM), jnp.float32),
        scratch_types=[
            pltpu.VMEM((NCHUNK, CHUNK), jnp.int32),
            pltpu.VMEM((NBUF, CHUNK, DIM_SEMANTIC), jnp.float32),
            pltpu.SemaphoreType.DMA((NBUF,)),
            pltpu.SemaphoreType.DMA((NBUF,)),
        ],
    )
    def sc_gather(ids_hbm, table_hbm, out_hbm, idx_v, rows_v, gsem, wsem):
        wid = lax.axis_index("s") * nc + lax.axis_index("c")
        base = wid * B_PER_W
        pltpu.sync_copy(ids_hbm.at[wid], idx_v)

        def gather(i, r):
            pltpu.async_copy(table_hbm.at[idx_v.at[i]], rows_v.at[r],
                             gsem.at[r])

        def wb_copy(i, r):
            return pltpu.make_async_copy(
                rows_v.at[r],
                out_hbm.at[pl.ds(base + i * CHUNK, CHUNK),
                           pl.ds(0, DIM_SEMANTIC)],
                wsem.at[r])

        for i in range(NBUF - 1):
            gather(i, i)

        def body(i, carry):
            r = lax.rem(i, NBUF)
            r2 = lax.rem(i + NBUF - 1, NBUF)
            pltpu.make_async_copy(rows_v.at[r],
                                  out_hbm.at[pl.ds(base, CHUNK),
                                             pl.ds(0, DIM_SEMANTIC)],
                                  gsem.at[r]).wait()
            wb_copy(i, r).start()

            @pl.when(jnp.logical_and(i >= 1, i + NBUF - 1 < NCHUNK))
            def _():
                wb_copy(i, r2).wait()

            @pl.when(i + NBUF - 1 < NCHUNK)
            def _():
                gather(i + NBUF - 1, r2)

            return carry

        lax.fori_loop(0, NCHUNK, body, 0)
        for r in range(NBUF):
            pltpu.make_async_copy(rows_v.at[r],
                                  out_hbm.at[pl.ds(base, CHUNK),
                                             pl.ds(0, DIM_SEMANTIC)],
                                  wsem.at[r]).wait()

    return sc_gather


def _pos_weights():
    m = np.zeros((32, 3), np.float32)
    ph = np.zeros((32, 1), np.float32)
    freqs = 2.0 ** np.linspace(0.0, MAX_FREQ_LOG2, NUM_FREQS)
    for i in range(NUM_FREQS):
        for s in range(2):
            for c in range(3):
                k = 6 * i + 3 * s + c
                m[k, c] = freqs[i]
                ph[k, 0] = 0.0 if s == 0 else np.pi / 2.0
    return jnp.asarray(m), jnp.asarray(ph)


_SIN_C = (6.2831852819, -41.341698212, 81.605064899, -76.702152496,
          42.020491157, -14.883436519, 3.2191201543)
_INV_2PI = 0.15915493667125702


def _fast_sin(t):
    u = t * jnp.float32(_INV_2PI)
    r = u - jnp.floor(u + jnp.float32(0.5))
    r2 = r * r
    acc = jnp.float32(_SIN_C[6])
    for k in range(5, -1, -1):
        acc = acc * r2 + jnp.float32(_SIN_C[k])
    return acc * r


def _dot_t(a, b):
    return lax.dot_general(a, b, (((0,), (0,)), ((), ())),
                           preferred_element_type=jnp.float32)


def _tc_body(sem_ref, coords_ref, inten_ref, m2_ref, ph_ref, wsin_ref,
             wsem_ref, wraw_ref, wi_ref, b_ref, out_ref):
    sem = sem_ref[:, :DIM_SEMANTIC]
    xt = coords_ref[...]
    t = lax.dot_general(m2_ref[...], xt, (((1,), (0,)), ((), ())),
                        preferred_element_type=jnp.float32,
                        precision=lax.Precision.HIGHEST)
    t = t + ph_ref[...]
    s = _fast_sin(t)
    acc = _dot_t(s, wsin_ref[...])
    acc = acc + jnp.dot(sem, wsem_ref[...],
                        preferred_element_type=jnp.float32)
    acc = acc + _dot_t(xt, wraw_ref[...])
    acc = acc + _dot_t(inten_ref[...], wi_ref[...])
    acc = acc + b_ref[...]
    out_ref[...] = acc


def kernel(coords, semantic_ids, intensity, embed_table, W, b):
    ids_pad = jnp.pad(semantic_ids.astype(jnp.int32), (0, NPAD - N))
    ids3 = ids_pad.reshape(NW, NCHUNK, CHUNK)
    sem_g = _sc_gather_fn()(ids3, embed_table)

    coords_t = coords.T
    inten_t = intensity.T

    cols = W.T
    m2, ph = _pos_weights()
    wsin = jnp.zeros((32, C_DIM), jnp.float32).at[:30].set(cols[3:33])
    wsem = cols[33:97]
    wraw = cols[0:3]
    wi = cols[97:98]
    b2 = b.reshape(1, C_DIM)

    grid = (N + BLK - 1) // BLK
    out = pl.pallas_call(
        _tc_body,
        grid=(grid,),
        in_specs=[
            pl.BlockSpec((BLK, C_DIM), lambda i: (i, 0)),
            pl.BlockSpec((3, BLK), lambda i: (0, i)),
            pl.BlockSpec((1, BLK), lambda i: (0, i)),
            pl.BlockSpec((32, 3), lambda i: (0, 0)),
            pl.BlockSpec((32, 1), lambda i: (0, 0)),
            pl.BlockSpec((32, C_DIM), lambda i: (0, 0)),
            pl.BlockSpec((64, C_DIM), lambda i: (0, 0)),
            pl.BlockSpec((3, C_DIM), lambda i: (0, 0)),
            pl.BlockSpec((1, C_DIM), lambda i: (0, 0)),
            pl.BlockSpec((1, C_DIM), lambda i: (0, 0)),
        ],
        out_specs=pl.BlockSpec((BLK, C_DIM), lambda i: (i, 0)),
        out_shape=jax.ShapeDtypeStruct((N, C_DIM), jnp.float32),
    )(sem_g, coords_t, inten_t, m2, ph, wsin, wsem, wraw, wi, b2)
    return out


# speedup vs baseline: 3.3097x; 2.0079x over previous
import functools

import jax
import jax.numpy as jnp
import numpy as np
from jax import lax
from jax.experimental import pallas as pl
from jax.experimental.pallas import tpu as pltpu
from jax.experimental.pallas import tpu_sc as plsc

N = 500000
NUM_SEMANTIC = 100000
DIM_SEMANTIC = 64
C_DIM = 128
NUM_FREQS = 5
MAX_FREQ_LOG2 = 4.0

NW = 32
CHUNK = 128
NBUF = 6

BLK = 4096
NBLK0 = 75
N0 = NBLK0 * BLK
NCHUNK0 = 75
N1 = N - N0
NCHUNK1 = 48
NBLK1 = 48


def _sc_gather_fn(nchunk):
    info = plsc.get_sparse_core_info()
    nc = info.num_cores
    b_per_w = CHUNK * nchunk
    npad = NW * b_per_w

    mesh = plsc.VectorSubcoreMesh(core_axis_name="c", subcore_axis_name="s")

    @functools.partial(
        pl.kernel,
        mesh=mesh,
        compiler_params=pltpu.CompilerParams(use_tc_tiling_on_sc=False),
        out_type=jax.ShapeDtypeStruct((npad, C_DIM), jnp.float32),
        scratch_types=[
            pltpu.VMEM((nchunk, CHUNK), jnp.int32),
            pltpu.VMEM((NBUF, CHUNK, DIM_SEMANTIC), jnp.float32),
            pltpu.SemaphoreType.DMA((NBUF,)),
            pltpu.SemaphoreType.DMA((NBUF,)),
        ],
    )
    def sc_gather(ids_hbm, table_hbm, out_hbm, idx_v, rows_v, gsem, wsem):
        wid = lax.axis_index("s") * nc + lax.axis_index("c")
        base = wid * b_per_w
        pltpu.sync_copy(ids_hbm.at[wid], idx_v)

        def gather(i, r):
            pltpu.async_copy(table_hbm.at[idx_v.at[i]], rows_v.at[r],
                             gsem.at[r])

        def wb_copy(i, r):
            return pltpu.make_async_copy(
                rows_v.at[r],
                out_hbm.at[pl.ds(base + i * CHUNK, CHUNK),
                           pl.ds(0, DIM_SEMANTIC)],
                wsem.at[r])

        for i in range(NBUF - 1):
            gather(i, i)

        def body(i, carry):
            r = lax.rem(i, NBUF)
            r2 = lax.rem(i + NBUF - 1, NBUF)
            pltpu.make_async_copy(rows_v.at[r],
                                  out_hbm.at[pl.ds(base, CHUNK),
                                             pl.ds(0, DIM_SEMANTIC)],
                                  gsem.at[r]).wait()
            wb_copy(i, r).start()

            @pl.when(jnp.logical_and(i >= 1, i + NBUF - 1 < nchunk))
            def _():
                wb_copy(i, r2).wait()

            @pl.when(i + NBUF - 1 < nchunk)
            def _():
                gather(i + NBUF - 1, r2)

            return carry

        lax.fori_loop(0, nchunk, body, 0)
        for r in range(NBUF):
            pltpu.make_async_copy(rows_v.at[r],
                                  out_hbm.at[pl.ds(base, CHUNK),
                                             pl.ds(0, DIM_SEMANTIC)],
                                  wsem.at[r]).wait()

    return sc_gather


def _pos_weights():
    m = np.zeros((32, 3), np.float32)
    ph = np.zeros((32, 1), np.float32)
    freqs = 2.0 ** np.linspace(0.0, MAX_FREQ_LOG2, NUM_FREQS)
    for i in range(NUM_FREQS):
        for s in range(2):
            for c in range(3):
                k = 6 * i + 3 * s + c
                m[k, c] = freqs[i]
                ph[k, 0] = 0.0 if s == 0 else np.pi / 2.0
    return jnp.asarray(m), jnp.asarray(ph)


_SIN_C = (6.2831852819, -41.341698212, 81.605064899, -76.702152496,
          42.020491157, -14.883436519, 3.2191201543)
_INV_2PI = 0.15915493667125702


def _fast_sin(t):
    u = t * jnp.float32(_INV_2PI)
    r = u - jnp.floor(u + jnp.float32(0.5))
    r2 = r * r
    acc = jnp.float32(_SIN_C[6])
    for k in range(5, -1, -1):
        acc = acc * r2 + jnp.float32(_SIN_C[k])
    return acc * r


def _dot_t(a, b):
    return lax.dot_general(a, b, (((0,), (0,)), ((), ())),
                           preferred_element_type=jnp.float32)


def _tc_body(sem_ref, coords_ref, inten_ref, m2_ref, ph_ref, wsin_ref,
             wsem_ref, wraw_ref, wi_ref, b_ref, out_ref):
    sem = sem_ref[:, :DIM_SEMANTIC]
    xt = coords_ref[...]
    t = lax.dot_general(m2_ref[...], xt, (((1,), (0,)), ((), ())),
                        preferred_element_type=jnp.float32,
                        precision=lax.Precision.HIGHEST)
    t = t + ph_ref[...]
    s = _fast_sin(t)
    acc = _dot_t(s, wsin_ref[...])
    acc = acc + jnp.dot(sem, wsem_ref[...],
                        preferred_element_type=jnp.float32)
    acc = acc + _dot_t(xt, wraw_ref[...])
    acc = acc + _dot_t(inten_ref[...], wi_ref[...])
    acc = acc + b_ref[...]
    out_ref[...] = acc


def _tc_body_alias(prev_ref, sem_ref, coords_ref, inten_ref, m2_ref, ph_ref,
                   wsin_ref, wsem_ref, wraw_ref, wi_ref, b_ref, out_ref):
    del prev_ref
    _tc_body(sem_ref, coords_ref, inten_ref, m2_ref, ph_ref, wsin_ref,
             wsem_ref, wraw_ref, wi_ref, b_ref, out_ref)


def kernel(coords, semantic_ids, intensity, embed_table, W, b):
    ids = semantic_ids.astype(jnp.int32)
    ids0 = ids[:N0].reshape(NW, NCHUNK0, CHUNK)
    ids1 = jnp.pad(ids[N0:], (0, NW * NCHUNK1 * CHUNK - N1))
    ids1 = ids1.reshape(NW, NCHUNK1, CHUNK)
    sem0 = _sc_gather_fn(NCHUNK0)(ids0, embed_table)
    sem1 = _sc_gather_fn(NCHUNK1)(ids1, embed_table)

    coords_t = coords.T
    inten_t = intensity.T

    cols = W.T
    m2, ph = _pos_weights()
    wsin = jnp.zeros((32, C_DIM), jnp.float32).at[:30].set(cols[3:33])
    wsem = cols[33:97]
    wraw = cols[0:3]
    wi = cols[97:98]
    b2 = b.reshape(1, C_DIM)

    common_specs = [
        pl.BlockSpec((3, BLK), lambda i: (0, i)),
        pl.BlockSpec((1, BLK), lambda i: (0, i)),
        pl.BlockSpec((32, 3), lambda i: (0, 0)),
        pl.BlockSpec((32, 1), lambda i: (0, 0)),
        pl.BlockSpec((32, C_DIM), lambda i: (0, 0)),
        pl.BlockSpec((64, C_DIM), lambda i: (0, 0)),
        pl.BlockSpec((3, C_DIM), lambda i: (0, 0)),
        pl.BlockSpec((1, C_DIM), lambda i: (0, 0)),
        pl.BlockSpec((1, C_DIM), lambda i: (0, 0)),
    ]
    weights = (m2, ph, wsin, wsem, wraw, wi, b2)

    out0 = pl.pallas_call(
        _tc_body,
        grid=(NBLK0,),
        in_specs=[pl.BlockSpec((BLK, C_DIM), lambda i: (i, 0))]
        + common_specs,
        out_specs=pl.BlockSpec((BLK, C_DIM), lambda i: (i, 0)),
        out_shape=jax.ShapeDtypeStruct((N, C_DIM), jnp.float32),
    )(sem0, coords_t, inten_t, *weights)

    out = pl.pallas_call(
        _tc_body_alias,
        grid=(NBLK1,),
        in_specs=[
            pl.BlockSpec(memory_space=pltpu.MemorySpace.HBM),
            pl.BlockSpec((BLK, C_DIM), lambda i: (i, 0)),
            pl.BlockSpec((3, BLK), lambda i: (0, i + NBLK0)),
            pl.BlockSpec((1, BLK), lambda i: (0, i + NBLK0)),
        ]
        + common_specs[2:],
        out_specs=pl.BlockSpec((BLK, C_DIM), lambda i: (i + NBLK0, 0)),
        out_shape=jax.ShapeDtypeStruct((N, C_DIM), jnp.float32),
        input_output_aliases={0: 0},
    )(out0, sem1, coords_t, inten_t, *weights)
    return out


# speedup vs baseline: 3.3797x; 1.0212x over previous
import functools

import jax
import jax.numpy as jnp
import numpy as np
from jax import lax
from jax.experimental import pallas as pl
from jax.experimental.pallas import tpu as pltpu
from jax.experimental.pallas import tpu_sc as plsc

N = 500000
NUM_SEMANTIC = 100000
DIM_SEMANTIC = 64
C_DIM = 128
NUM_FREQS = 5
MAX_FREQ_LOG2 = 4.0

NW = 32
CHUNK = 128
NBUF = 6

BLK = 4096
PARTS = (12, 55, 56)


def _sc_gather_fn(nchunk):
    info = plsc.get_sparse_core_info()
    nc = info.num_cores
    b_per_w = CHUNK * nchunk
    npad = NW * b_per_w

    mesh = plsc.VectorSubcoreMesh(core_axis_name="c", subcore_axis_name="s")

    @functools.partial(
        pl.kernel,
        mesh=mesh,
        compiler_params=pltpu.CompilerParams(use_tc_tiling_on_sc=False),
        out_type=jax.ShapeDtypeStruct((npad, C_DIM), jnp.float32),
        scratch_types=[
            pltpu.VMEM((nchunk, CHUNK), jnp.int32),
            pltpu.VMEM((NBUF, CHUNK, DIM_SEMANTIC), jnp.float32),
            pltpu.SemaphoreType.DMA((NBUF,)),
            pltpu.SemaphoreType.DMA((NBUF,)),
        ],
    )
    def sc_gather(ids_hbm, table_hbm, out_hbm, idx_v, rows_v, gsem, wsem):
        wid = lax.axis_index("s") * nc + lax.axis_index("c")
        base = wid * b_per_w
        pltpu.sync_copy(ids_hbm.at[wid], idx_v)

        def gather(i, r):
            pltpu.async_copy(table_hbm.at[idx_v.at[i]], rows_v.at[r],
                             gsem.at[r])

        def wb_copy(i, r):
            return pltpu.make_async_copy(
                rows_v.at[r],
                out_hbm.at[pl.ds(base + i * CHUNK, CHUNK),
                           pl.ds(0, DIM_SEMANTIC)],
                wsem.at[r])

        for i in range(NBUF - 1):
            gather(i, i)

        def body(i, carry):
            r = lax.rem(i, NBUF)
            r2 = lax.rem(i + NBUF - 1, NBUF)
            pltpu.make_async_copy(rows_v.at[r],
                                  out_hbm.at[pl.ds(base, CHUNK),
                                             pl.ds(0, DIM_SEMANTIC)],
                                  gsem.at[r]).wait()
            wb_copy(i, r).start()

            @pl.when(jnp.logical_and(i >= 1, i + NBUF - 1 < nchunk))
            def _():
                wb_copy(i, r2).wait()

            @pl.when(i + NBUF - 1 < nchunk)
            def _():
                gather(i + NBUF - 1, r2)

            return carry

        lax.fori_loop(0, nchunk, body, 0)
        for r in range(NBUF):
            pltpu.make_async_copy(rows_v.at[r],
                                  out_hbm.at[pl.ds(base, CHUNK),
                                             pl.ds(0, DIM_SEMANTIC)],
                                  wsem.at[r]).wait()

    return sc_gather


def _pos_weights():
    m = np.zeros((32, 3), np.float32)
    ph = np.zeros((32, 1), np.float32)
    freqs = 2.0 ** np.linspace(0.0, MAX_FREQ_LOG2, NUM_FREQS)
    for i in range(NUM_FREQS):
        for s in range(2):
            for c in range(3):
                k = 6 * i + 3 * s + c
                m[k, c] = freqs[i]
                ph[k, 0] = 0.0 if s == 0 else np.pi / 2.0
    return jnp.asarray(m), jnp.asarray(ph)


_SIN_C = (6.2831852819, -41.341698212, 81.605064899, -76.702152496,
          42.020491157, -14.883436519, 3.2191201543)
_INV_2PI = 0.15915493667125702


def _fast_sin(t):
    u = t * jnp.float32(_INV_2PI)
    r = u - jnp.floor(u + jnp.float32(0.5))
    r2 = r * r
    acc = jnp.float32(_SIN_C[6])
    for k in range(5, -1, -1):
        acc = acc * r2 + jnp.float32(_SIN_C[k])
    return acc * r


def _dot_t(a, b):
    return lax.dot_general(a, b, (((0,), (0,)), ((), ())),
                           preferred_element_type=jnp.float32)


def _tc_body(sem_ref, coords_ref, inten_ref, m2_ref, ph_ref, wsin_ref,
             wsem_ref, wraw_ref, wi_ref, b_ref, out_ref):
    sem = sem_ref[:, :DIM_SEMANTIC]
    xt = coords_ref[...]
    t = lax.dot_general(m2_ref[...], xt, (((1,), (0,)), ((), ())),
                        preferred_element_type=jnp.float32,
                        precision=lax.Precision.HIGHEST)
    t = t + ph_ref[...]
    s = _fast_sin(t)
    acc = _dot_t(s, wsin_ref[...])
    acc = acc + jnp.dot(sem, wsem_ref[...],
                        preferred_element_type=jnp.float32)
    acc = acc + _dot_t(xt, wraw_ref[...])
    acc = acc + _dot_t(inten_ref[...], wi_ref[...])
    acc = acc + b_ref[...]
    out_ref[...] = acc


def _tc_body_alias(prev_ref, sem_ref, coords_ref, inten_ref, m2_ref, ph_ref,
                   wsin_ref, wsem_ref, wraw_ref, wi_ref, b_ref, out_ref):
    del prev_ref
    _tc_body(sem_ref, coords_ref, inten_ref, m2_ref, ph_ref, wsin_ref,
             wsem_ref, wraw_ref, wi_ref, b_ref, out_ref)


def kernel(coords, semantic_ids, intensity, embed_table, W, b):
    ids = semantic_ids.astype(jnp.int32)
    sems = []
    off = 0
    for nblk in PARTS:
        start, stop = off * BLK, min((off + nblk) * BLK, N)
        npad = NW * nblk * CHUNK
        part = ids[start:stop]
        if stop - start < npad:
            part = jnp.pad(part, (0, npad - (stop - start)))
        sems.append(_sc_gather_fn(nblk)(part.reshape(NW, nblk, CHUNK),
                                        embed_table))
        off += nblk

    coords_t = coords.T
    inten_t = intensity.T

    cols = W.T
    m2, ph = _pos_weights()
    wsin = jnp.zeros((32, C_DIM), jnp.float32).at[:30].set(cols[3:33])
    wsem = cols[33:97]
    wraw = cols[0:3]
    wi = cols[97:98]
    b2 = b.reshape(1, C_DIM)

    common_specs = [
        pl.BlockSpec((3, BLK), lambda i: (0, i)),
        pl.BlockSpec((1, BLK), lambda i: (0, i)),
        pl.BlockSpec((32, 3), lambda i: (0, 0)),
        pl.BlockSpec((32, 1), lambda i: (0, 0)),
        pl.BlockSpec((32, C_DIM), lambda i: (0, 0)),
        pl.BlockSpec((64, C_DIM), lambda i: (0, 0)),
        pl.BlockSpec((3, C_DIM), lambda i: (0, 0)),
        pl.BlockSpec((1, C_DIM), lambda i: (0, 0)),
        pl.BlockSpec((1, C_DIM), lambda i: (0, 0)),
    ]
    weights = (m2, ph, wsin, wsem, wraw, wi, b2)
    out_shape = jax.ShapeDtypeStruct((N, C_DIM), jnp.float32)

    out = None
    off = 0
    for k, nblk in enumerate(PARTS):
        o = off
        part_specs = [
            pl.BlockSpec((BLK, C_DIM), lambda i: (i, 0)),
            pl.BlockSpec((3, BLK), lambda i, o=o: (0, i + o)),
            pl.BlockSpec((1, BLK), lambda i, o=o: (0, i + o)),
        ] + common_specs[2:]
        out_spec = pl.BlockSpec((BLK, C_DIM), lambda i, o=o: (i + o, 0))
        if k == 0:
            out = pl.pallas_call(
                _tc_body,
                grid=(nblk,),
                in_specs=part_specs,
                out_specs=out_spec,
                out_shape=out_shape,
            )(sems[k], coords_t, inten_t, *weights)
        else:
            out = pl.pallas_call(
                _tc_body_alias,
                grid=(nblk,),
                in_specs=[pl.BlockSpec(memory_space=pltpu.MemorySpace.HBM)]
                + part_specs,
                out_specs=out_spec,
                out_shape=out_shape,
                input_output_aliases={0: 0},
            )(out, sems[k], coords_t, inten_t, *weights)
        off += nblk
    return out
